# Initial kernel scaffold; baseline (speedup 1.0000x reference)
import jax
import jax.numpy as jnp
from jax.experimental import pallas as pl


def kernel(V, F, int_edge_vertices, edge_tri_indices):
    raise NotImplementedError("write your pallas kernel here")


# speedup vs baseline: 356.2130x; 356.2130x over previous
import functools

import jax
import jax.numpy as jnp
from jax import lax
from jax.experimental import pallas as pl
from jax.experimental.pallas import tpu as pltpu
from jax.experimental.pallas import tpu_sc as plsc

NC = 2
NS = 16
NW = NC * NS
L = 16


def _pad_to(n, m):
    return ((n + m - 1) // m) * m


def _wid():
    return lax.axis_index("s") * NC + lax.axis_index("c")


def _iota16():
    return lax.iota(jnp.int32, L)


def _full16(c):
    return jnp.full((L,), c, jnp.int32)


def _col(ref, rows, c):
    return plsc.load_gather(ref, [rows, _full16(c)])


_SC_PARAMS = pltpu.CompilerParams(needs_layout_passes=False,
                                 use_tc_tiling_on_sc=False)


def _make_tri_kernel(NV, T_pad, B):
    C = T_pad // NW
    nblk = C // B
    mesh = plsc.VectorSubcoreMesh(core_axis_name="c", subcore_axis_name="s",
                                  num_cores=NC, num_subcores=NS)

    @functools.partial(
        pl.kernel,
        out_type=jax.ShapeDtypeStruct((T_pad, 8), jnp.float32),
        mesh=mesh,
        compiler_params=_SC_PARAMS,
        scratch_types=[
            pltpu.VMEM((B,), jnp.int32),
            pltpu.VMEM((B,), jnp.int32),
            pltpu.VMEM((B,), jnp.int32),
            pltpu.VMEM((B, 8), jnp.float32),
            pltpu.VMEM((B, 8), jnp.float32),
            pltpu.VMEM((B, 8), jnp.float32),
            pltpu.VMEM((B, 8), jnp.float32),
            pltpu.SemaphoreType.DMA,
            pltpu.SemaphoreType.DMA,
            pltpu.SemaphoreType.DMA,
        ],
    )
    def tri_kernel(v_hbm, ft_hbm, s_hbm, i0, i1, i2, r0, r1, r2, so,
                   sem0, sem1, sem2):
        wbase = _wid() * C
        for blk in range(nblk):
            base = wbase + blk * B
            pltpu.sync_copy(ft_hbm.at[pl.ds(0 * T_pad + base, B)], i0)
            pltpu.sync_copy(ft_hbm.at[pl.ds(1 * T_pad + base, B)], i1)
            pltpu.sync_copy(ft_hbm.at[pl.ds(2 * T_pad + base, B)], i2)
            c0 = pltpu.async_copy(v_hbm.at[i0], r0, sem0)
            c1 = pltpu.async_copy(v_hbm.at[i1], r1, sem1)
            c2 = pltpu.async_copy(v_hbm.at[i2], r2, sem2)
            c0.wait()
            c1.wait()
            c2.wait()
            _tri_block(r0, r1, r2, so, B)
            pltpu.sync_copy(so, s_hbm.at[pl.ds(base, B)])

    return tri_kernel


def _tri_block(r0, r1, r2, so, B):
        def body(i, _):
            rows = i * L + _iota16()
            x0 = _col(r0, rows, 0)
            y0 = _col(r0, rows, 1)
            z0 = _col(r0, rows, 2)
            ux = _col(r1, rows, 0) - x0
            uy = _col(r1, rows, 1) - y0
            uz = _col(r1, rows, 2) - z0
            wx = _col(r2, rows, 0) - x0
            wy = _col(r2, rows, 1) - y0
            wz = _col(r2, rows, 2) - z0
            uu = ux * ux + uy * uy + uz * uz
            uv = ux * wx + uy * wy + uz * wz
            vv = wx * wx + wy * wy + wz * wz
            inv = 1.0 / (uu * vv - uv * uv)
            a = (vv * uz - uv * wz) * inv
            b = (uu * wz - uv * uz) * inv
            plsc.store_scatter(so, [rows, _full16(0)], -(a * ux + b * wx))
            plsc.store_scatter(so, [rows, _full16(1)], -(a * uy + b * wy))
            plsc.store_scatter(so, [rows, _full16(2)], -(a * uz + b * wz))
            return _

        lax.fori_loop(0, B // L, body, None)


def _make_edge_kernel(NV, T_pad, E_pad, B):
    C = E_pad // NW
    nblk = C // B
    mesh = plsc.VectorSubcoreMesh(core_axis_name="c", subcore_axis_name="s",
                                  num_cores=NC, num_subcores=NS)
    out = jax.ShapeDtypeStruct((E_pad,), jnp.float32)

    @functools.partial(
        pl.kernel,
        out_type=(out, out, out),
        mesh=mesh,
        compiler_params=_SC_PARAMS,
        scratch_types=(
            [pltpu.VMEM((B,), jnp.int32) for _ in range(6)]
            + [pltpu.VMEM((B, 8), jnp.float32) for _ in range(6)]
            + [pltpu.VMEM((B,), jnp.float32) for _ in range(3)]
            + [pltpu.SemaphoreType.DMA for _ in range(6)]
        ),
    )
    def edge_kernel(v_hbm, s_hbm, ev_hbm, et_hbm, d1_hbm, d2_hbm, fl_hbm,
                    ia, ib, ic, id_, it1, it2, ra, rb, rc, rd, r1, r2,
                    d1, d2, fl, s0, s1, s2, s3, s4, s5):
        wbase = _wid() * C
        for blk in range(nblk):
            off = wbase + blk * B
            pltpu.sync_copy(ev_hbm.at[pl.ds(0 * E_pad + off, B)], ia)
            pltpu.sync_copy(ev_hbm.at[pl.ds(1 * E_pad + off, B)], ib)
            pltpu.sync_copy(ev_hbm.at[pl.ds(2 * E_pad + off, B)], ic)
            pltpu.sync_copy(ev_hbm.at[pl.ds(3 * E_pad + off, B)], id_)
            pltpu.sync_copy(et_hbm.at[pl.ds(0 * E_pad + off, B)], it1)
            pltpu.sync_copy(et_hbm.at[pl.ds(1 * E_pad + off, B)], it2)
            cps = [
                pltpu.async_copy(v_hbm.at[ia], ra, s0),
                pltpu.async_copy(v_hbm.at[ib], rb, s1),
                pltpu.async_copy(v_hbm.at[ic], rc, s2),
                pltpu.async_copy(v_hbm.at[id_], rd, s3),
                pltpu.async_copy(s_hbm.at[it1], r1, s4),
                pltpu.async_copy(s_hbm.at[it2], r2, s5),
            ]
            for cp in cps:
                cp.wait()

            def body(i, _):
                rows = i * L + _iota16()
                ax = _col(ra, rows, 0)
                ay = _col(ra, rows, 1)
                az = _col(ra, rows, 2)
                abx = _col(rb, rows, 0) - ax
                aby = _col(rb, rows, 1) - ay
                abz = _col(rb, rows, 2) - az
                acx = _col(rc, rows, 0) - ax
                acy = _col(rc, rows, 1) - ay
                acz = _col(rc, rows, 2) - az
                adx = _col(rd, rows, 0) - ax
                ady = _col(rd, rows, 1) - ay
                adz = _col(rd, rows, 2) - az
                den = jnp.maximum(abx * abx + aby * aby + abz * abz, 1e-8)
                rden = 1.0 / den
                p1 = (acx * abx + acy * aby + acz * abz) * rden
                p2 = (adx * abx + ady * aby + adz * abz) * rden
                h1x = acx - p1 * abx
                h1y = acy - p1 * aby
                h1z = acz - p1 * abz
                h2x = adx - p2 * abx
                h2y = ady - p2 * aby
                h2z = adz - p2 * abz
                s1x = _col(r1, rows, 0)
                s1y = _col(r1, rows, 1)
                s1z = _col(r1, rows, 2)
                s2x = _col(r2, rows, 0)
                s2y = _col(r2, rows, 1)
                s2z = _col(r2, rows, 2)
                dot1 = s1x * h1x + s1y * h1y + s1z * h1z
                dot2 = s2x * h2x + s2y * h2y + s2z * h2z
                sl = pl.ds(i * L, L)
                d1[sl] = dot1
                d2[sl] = dot2
                fl[sl] = jnp.where((dot1 > 0.0) & (dot2 > 0.0), 1.0, 0.0)
                return _

            lax.fori_loop(0, B // L, body, None)
            pltpu.sync_copy(d1, d1_hbm.at[pl.ds(off, B)])
            pltpu.sync_copy(d2, d2_hbm.at[pl.ds(off, B)])
            pltpu.sync_copy(fl, fl_hbm.at[pl.ds(off, B)])

    return edge_kernel


@jax.jit
def kernel(V, F, int_edge_vertices, edge_tri_indices):
    NV = V.shape[0]
    T = F.shape[0]
    E = int_edge_vertices.shape[0]

    B_t = 1360
    B_e = 1536
    T_pad = _pad_to(T, NW * B_t)
    E_pad = _pad_to(E, NW * B_e)

    ft = jnp.pad(F.astype(jnp.int32), ((0, T_pad - T), (0, 0))).T.reshape(-1)
    ev = jnp.pad(int_edge_vertices.astype(jnp.int32),
                 ((0, E_pad - E), (0, 0))).T.reshape(-1)
    et = jnp.pad(edge_tri_indices.astype(jnp.int32),
                 ((0, E_pad - E), (0, 0))).T.reshape(-1)
    V8 = jnp.pad(V, ((0, 0), (0, 5)))

    s_tris = _make_tri_kernel(NV, T_pad, B_t)(V8, ft)
    d1, d2, fl = _make_edge_kernel(NV, T_pad, E_pad, B_e)(V8, s_tris, ev, et)
    return (fl[:E].astype(bool), d1[:E], d2[:E])


# speedup vs baseline: 436.9394x; 1.2266x over previous
import functools

import jax
import jax.numpy as jnp
from jax import lax
from jax.experimental import pallas as pl
from jax.experimental.pallas import tpu as pltpu
from jax.experimental.pallas import tpu_sc as plsc

NC = 2
NS = 16
NW = NC * NS
L = 16
W = 8


def _pad_to(n, m):
    return ((n + m - 1) // m) * m


def _wid():
    return lax.axis_index("s") * NC + lax.axis_index("c")


def _iota16():
    return lax.iota(jnp.int32, L)


def _full16(c):
    return jnp.full((L,), c, jnp.int32)


def _col(ref, rows, c):
    return plsc.load_gather(ref, [rows, _full16(c)])


_SC_PARAMS = pltpu.CompilerParams(needs_layout_passes=False,
                                  use_tc_tiling_on_sc=False)

_MESH = plsc.VectorSubcoreMesh(core_axis_name="c", subcore_axis_name="s",
                               num_cores=NC, num_subcores=NS)


def _make_tri_kernel(T_pad, B):
    C = T_pad // NW
    nblk = C // B

    @functools.partial(
        pl.kernel,
        out_type=jax.ShapeDtypeStruct((T_pad, W), jnp.float32),
        mesh=_MESH,
        compiler_params=_SC_PARAMS,
        scratch_types=(
            [pltpu.VMEM((C,), jnp.int32) for _ in range(3)]
            + [pltpu.VMEM((B, W), jnp.float32) for _ in range(6)]
            + [pltpu.VMEM((B, W), jnp.float32)]
            + [pltpu.SemaphoreType.DMA for _ in range(6)]
        ),
    )
    def tri_kernel(v_hbm, ft_hbm, s_hbm, i0, i1, i2,
                   r0a, r1a, r2a, r0b, r1b, r2b, so, *sems):
        wbase = _wid() * C
        pltpu.sync_copy(ft_hbm.at[pl.ds(0 * T_pad + wbase, C)], i0)
        pltpu.sync_copy(ft_hbm.at[pl.ds(1 * T_pad + wbase, C)], i1)
        pltpu.sync_copy(ft_hbm.at[pl.ds(2 * T_pad + wbase, C)], i2)
        bufs = ((r0a, r1a, r2a), (r0b, r1b, r2b))

        def fire(blk, grp):
            o = blk * B
            rs = bufs[grp]
            ss = sems[3 * grp:3 * grp + 3]
            return [
                pltpu.async_copy(v_hbm.at[i0.at[pl.ds(o, B)]], rs[0], ss[0]),
                pltpu.async_copy(v_hbm.at[i1.at[pl.ds(o, B)]], rs[1], ss[1]),
                pltpu.async_copy(v_hbm.at[i2.at[pl.ds(o, B)]], rs[2], ss[2]),
            ]

        pend = fire(0, 0)
        for blk in range(nblk):
            cur = pend
            if blk + 1 < nblk:
                pend = fire(blk + 1, (blk + 1) % 2)
            for cp in cur:
                cp.wait()
            r0, r1, r2 = bufs[blk % 2]

            def body(i, _):
                rows = i * L + _iota16()
                x0 = _col(r0, rows, 0)
                y0 = _col(r0, rows, 1)
                z0 = _col(r0, rows, 2)
                ux = _col(r1, rows, 0) - x0
                uy = _col(r1, rows, 1) - y0
                uz = _col(r1, rows, 2) - z0
                wx = _col(r2, rows, 0) - x0
                wy = _col(r2, rows, 1) - y0
                wz = _col(r2, rows, 2) - z0
                uu = ux * ux + uy * uy + uz * uz
                uv = ux * wx + uy * wy + uz * wz
                vv = wx * wx + wy * wy + wz * wz
                inv = 1.0 / (uu * vv - uv * uv)
                a = (vv * uz - uv * wz) * inv
                b = (uu * wz - uv * uz) * inv
                plsc.store_scatter(so, [rows, _full16(0)], -(a * ux + b * wx))
                plsc.store_scatter(so, [rows, _full16(1)], -(a * uy + b * wy))
                plsc.store_scatter(so, [rows, _full16(2)], -(a * uz + b * wz))
                return _

            lax.fori_loop(0, B // L, body, None)
            pltpu.sync_copy(so, s_hbm.at[pl.ds(wbase + blk * B, B)])

    return tri_kernel


def _make_edge_kernel(T_pad, E_pad, B):
    C = E_pad // NW
    nblk = C // B
    out = jax.ShapeDtypeStruct((E_pad,), jnp.float32)

    @functools.partial(
        pl.kernel,
        out_type=(out, out, out),
        mesh=_MESH,
        compiler_params=_SC_PARAMS,
        scratch_types=(
            [pltpu.VMEM((C,), jnp.int32) for _ in range(6)]
            + [pltpu.VMEM((B, W), jnp.float32) for _ in range(12)]
            + [pltpu.VMEM((B,), jnp.float32) for _ in range(3)]
            + [pltpu.SemaphoreType.DMA for _ in range(12)]
        ),
    )
    def edge_kernel(v_hbm, s_hbm, ev_hbm, et_hbm, d1_hbm, d2_hbm, fl_hbm,
                    ia, ib, ic, id_, it1, it2,
                    raa, rba, rca, rda, r1a, r2a,
                    rab, rbb, rcb, rdb, r1b, r2b,
                    d1, d2, fl, *sems):
        wbase = _wid() * C
        pltpu.sync_copy(ev_hbm.at[pl.ds(0 * E_pad + wbase, C)], ia)
        pltpu.sync_copy(ev_hbm.at[pl.ds(1 * E_pad + wbase, C)], ib)
        pltpu.sync_copy(ev_hbm.at[pl.ds(2 * E_pad + wbase, C)], ic)
        pltpu.sync_copy(ev_hbm.at[pl.ds(3 * E_pad + wbase, C)], id_)
        pltpu.sync_copy(et_hbm.at[pl.ds(0 * E_pad + wbase, C)], it1)
        pltpu.sync_copy(et_hbm.at[pl.ds(1 * E_pad + wbase, C)], it2)
        bufs = ((raa, rba, rca, rda, r1a, r2a),
                (rab, rbb, rcb, rdb, r1b, r2b))
        idxs = (ia, ib, ic, id_, it1, it2)

        def fire(blk, grp):
            o = blk * B
            rs = bufs[grp]
            ss = sems[6 * grp:6 * grp + 6]
            cps = []
            for j in range(4):
                cps.append(pltpu.async_copy(
                    v_hbm.at[idxs[j].at[pl.ds(o, B)]], rs[j], ss[j]))
            for j in (4, 5):
                cps.append(pltpu.async_copy(
                    s_hbm.at[idxs[j].at[pl.ds(o, B)]], rs[j], ss[j]))
            return cps

        pend = fire(0, 0)
        for blk in range(nblk):
            cur = pend
            if blk + 1 < nblk:
                pend = fire(blk + 1, (blk + 1) % 2)
            for cp in cur:
                cp.wait()
            ra, rb, rc, rd, r1, r2 = bufs[blk % 2]

            def body(i, _):
                rows = i * L + _iota16()
                ax = _col(ra, rows, 0)
                ay = _col(ra, rows, 1)
                az = _col(ra, rows, 2)
                abx = _col(rb, rows, 0) - ax
                aby = _col(rb, rows, 1) - ay
                abz = _col(rb, rows, 2) - az
                acx = _col(rc, rows, 0) - ax
                acy = _col(rc, rows, 1) - ay
                acz = _col(rc, rows, 2) - az
                adx = _col(rd, rows, 0) - ax
                ady = _col(rd, rows, 1) - ay
                adz = _col(rd, rows, 2) - az
                den = jnp.maximum(abx * abx + aby * aby + abz * abz, 1e-8)
                rden = 1.0 / den
                p1 = (acx * abx + acy * aby + acz * abz) * rden
                p2 = (adx * abx + ady * aby + adz * abz) * rden
                h1x = acx - p1 * abx
                h1y = acy - p1 * aby
                h1z = acz - p1 * abz
                h2x = adx - p2 * abx
                h2y = ady - p2 * aby
                h2z = adz - p2 * abz
                s1x = _col(r1, rows, 0)
                s1y = _col(r1, rows, 1)
                s1z = _col(r1, rows, 2)
                s2x = _col(r2, rows, 0)
                s2y = _col(r2, rows, 1)
                s2z = _col(r2, rows, 2)
                dot1 = s1x * h1x + s1y * h1y + s1z * h1z
                dot2 = s2x * h2x + s2y * h2y + s2z * h2z
                sl = pl.ds(i * L, L)
                d1[sl] = dot1
                d2[sl] = dot2
                fl[sl] = jnp.where((dot1 > 0.0) & (dot2 > 0.0), 1.0, 0.0)
                return _

            lax.fori_loop(0, B // L, body, None)
            off = wbase + blk * B
            pltpu.sync_copy(d1, d1_hbm.at[pl.ds(off, B)])
            pltpu.sync_copy(d2, d2_hbm.at[pl.ds(off, B)])
            pltpu.sync_copy(fl, fl_hbm.at[pl.ds(off, B)])

    return edge_kernel


@jax.jit
def kernel(V, F, int_edge_vertices, edge_tri_indices):
    T = F.shape[0]
    E = int_edge_vertices.shape[0]

    B_t = 816
    B_e = 768
    T_pad = _pad_to(T, NW * B_t)
    E_pad = _pad_to(E, NW * B_e)

    ft = jnp.pad(F.astype(jnp.int32), ((0, T_pad - T), (0, 0))).T.reshape(-1)
    ev = jnp.pad(int_edge_vertices.astype(jnp.int32),
                 ((0, E_pad - E), (0, 0))).T.reshape(-1)
    et = jnp.pad(edge_tri_indices.astype(jnp.int32),
                 ((0, E_pad - E), (0, 0))).T.reshape(-1)
    V8 = jnp.pad(V, ((0, 0), (0, W - 3)))

    s_tris = _make_tri_kernel(T_pad, B_t)(V8, ft)
    d1, d2, fl = _make_edge_kernel(T_pad, E_pad, B_e)(V8, s_tris, ev, et)
    return (fl[:E].astype(bool), d1[:E], d2[:E])
